# baseline (speedup 1.0000x reference)
import functools

import jax
import jax.numpy as jnp
from jax import lax
from jax.experimental import pallas as pl
from jax.experimental.pallas import tpu as pltpu
from jax.experimental.pallas import tpu_sc as plsc

N = 10000
E = 160000
COORS = 2
D_IN = 256
H = 256
D_OUT = 256
NC = 2
NS = 16
L = 16
HH = H // NC
EPW = E // NS
C = 80
NCHUNK = EPW // C
STRIP = 624
VPE = HH // L


_BM = 2000


def _mm_a_body(x_ref, w_ref, pos_ref, reg_ref, o_ref, ptab_ref):
    o_ref[...] = jnp.dot(x_ref[...], w_ref[...],
                         preferred_element_type=jnp.float32)

    @pl.when(pl.program_id(1) == 0)
    def _():
        q = jnp.clip((pos_ref[...] * QSCALE).astype(jnp.int32),
                     0, QSCALE - 1)
        qx, qy = q[:, 0:1], q[:, 1:2]
        ptab_ref[...] = (qx << 18) | (qy << 4) | (reg_ref[...] & 0xF)


def _matmul_a(x, w_lin, pos, region):
    nb = N // _BM
    return pl.pallas_call(
        _mm_a_body,
        grid=(nb, NC),
        in_specs=[
            pl.BlockSpec((_BM, D_IN), lambda i, h: (i, 0)),
            pl.BlockSpec((D_IN, HH), lambda i, h: (0, h)),
            pl.BlockSpec((_BM, COORS), lambda i, h: (i, 0)),
            pl.BlockSpec((_BM, 1), lambda i, h: (i, 0)),
        ],
        out_specs=[
            pl.BlockSpec((_BM, HH), lambda i, h: (h * nb + i, 0)),
            pl.BlockSpec((_BM, 1), lambda i, h: (i, 0)),
        ],
        out_shape=[
            jax.ShapeDtypeStruct((NC * N, HH), jnp.float32),
            jax.ShapeDtypeStruct((N, 1), jnp.int32),
        ],
    )(x, w_lin, pos, region[:, None])


def _mm_c_body(a0_ref, a1_ref, w0_ref, w1_ref, b_ref, o_ref):
    acc = jnp.dot(a0_ref[...], w0_ref[...], preferred_element_type=jnp.float32)
    acc += jnp.dot(a1_ref[...], w1_ref[...], preferred_element_type=jnp.float32)
    o_ref[...] = acc + b_ref[...]


def _matmul_c(aflat, w_out, b_out2):
    nb = N // _BM
    return pl.pallas_call(
        _mm_c_body,
        grid=(nb,),
        in_specs=[
            pl.BlockSpec((_BM, HH), lambda i: (i, 0)),
            pl.BlockSpec((_BM, HH), lambda i: (nb + i, 0)),
            pl.BlockSpec((HH, D_OUT), lambda i: (0, 0)),
            pl.BlockSpec((HH, D_OUT), lambda i: (1, 0)),
            pl.BlockSpec((1, D_OUT), lambda i: (0, 0)),
        ],
        out_specs=pl.BlockSpec((_BM, D_OUT), lambda i: (i, 0)),
        out_shape=jax.ShapeDtypeStruct((N, D_OUT), jnp.float32),
    )(aflat, aflat, w_out, w_out, b_out2)


def _splat(v):
    return lax.broadcast(v, (L,))


def _sc_body(hflat, eidx_hbm, ptab_hbm,
             wpos_hbm, bpos_hbm,
             aflat,
             wpos_v, bpos_v, ptab_v,
             ijbufA, adjA, rxA, ryA, mdA,
             ijbufB, adjB, rxB, ryB, mdB,
             sdix0, sdix1, hbufA, hbufB, msgbuf, acc,
             semA, semB, semS0, semS1, semIA, semIB):
    c = lax.axis_index("c")
    s = lax.axis_index("s")
    cN = c * N

    pltpu.sync_copy(wpos_hbm, wpos_v)
    pltpu.sync_copy(bpos_hbm, bpos_v)
    pltpu.sync_copy(ptab_hbm, ptab_v)

    coff = c * HH
    w0 = [wpos_v[0, pl.ds(coff + v * L, L)] for v in range(VPE)]
    w1 = [wpos_v[1, pl.ds(coff + v * L, L)] for v in range(VPE)]
    bb = [bpos_v[pl.ds(coff + v * L, L)] for v in range(VPE)]

    zero16 = jnp.zeros((L,), jnp.float32)

    def _zero_msg(r, _):
        for v in range(VPE):
            msgbuf[r, pl.ds(v * L, L)] = zero16
        return _

    lax.fori_loop(0, C, _zero_msg, 0, unroll=False)
    base_n = s * STRIP
    for r in range(8):
        pltpu.sync_copy(msgbuf, acc.at[pl.ds(base_n + r * C, C)])
    plsc.subcore_barrier()

    ebase = s * EPW
    inv_q = jnp.float32(1.0 / QSCALE)

    def _wait_scatter0():
        pltpu.make_async_copy(msgbuf.at[pl.ds(0, CH0)],
                              acc.at[sdix0], semS0).wait()

    def _wait_scatter1():
        pltpu.make_async_copy(msgbuf.at[pl.ds(CH0, CH1)],
                              acc.at[sdix1], semS1).wait()

    def _issue_idx(j, ijbufX, semIX):
        off = s * EPW + jnp.minimum(j, NCHUNK - 1) * C
        pltpu.async_copy(eidx_hbm.at[pl.ds(off, C)],
                         ijbufX.at[pl.ds(0, C)], semIX)
        pltpu.async_copy(eidx_hbm.at[pl.ds(E + off, C)],
                         ijbufX.at[pl.ds(C, C)], semIX)

    def _wait_idx(ijbufX, semIX):
        pltpu.make_async_copy(eidx_hbm.at[pl.ds(0, 2 * C)], ijbufX,
                              semIX).wait()

    def _prefetch(j, ijbufX, adjX, rxX, ryX, mdX, hbufX, semX, semIX):
        _wait_idx(ijbufX, semIX)
        for g in range(C // L):
            sl = pl.ds(g * L, L)
            adjX[sl] = ijbufX[sl] + cN
        pltpu.async_copy(hflat.at[adjX], hbufX, semX)
        for g in range(C // L):
            sl = pl.ds(g * L, L)
            ps = plsc.load_gather(ptab_v, [ijbufX[sl]])
            pd = plsc.load_gather(ptab_v, [ijbufX[pl.ds(C + g * L, L)]])
            qxs = lax.shift_right_logical(ps, 18)
            qxd = lax.shift_right_logical(pd, 18)
            qys = lax.shift_right_logical(ps, 4) & 0x3FFF
            qyd = lax.shift_right_logical(pd, 4) & 0x3FFF
            rxX[sl] = (qxs - qxd).astype(jnp.float32) * inv_q
            ryX[sl] = (qys - qyd).astype(jnp.float32) * inv_q
            mdX[sl] = jnp.where((ps & 0xF) == (pd & 0xF),
                                ijbufX[pl.ds(C + g * L, L)], N + s)
        _issue_idx(j + 2, ijbufX, semIX)

    def _compute(wait_prev, ijbufX, adjX, rxX, ryX, mdX, hbufX, semX, semIX):
        pltpu.make_async_copy(hflat.at[adjX], hbufX, semX).wait()

        def _edges(lo, hi):
            @plsc.parallel_loop(lo, hi, unroll=4)
            def _edge(e):
                ev = _splat(e)
                rxv = plsc.load_gather(rxX, [ev])
                ryv = plsc.load_gather(ryX, [ev])
                for v in range(VPE):
                    sl = pl.ds(v * L, L)
                    h16 = hbufX[e, sl]
                    z = rxv * w0[v] + ryv * w1[v] + bb[v]
                    gate = jnp.maximum(z, 0.0)
                    msgbuf[e, sl] = gate * h16

        @pl.when(wait_prev)
        def _():
            _wait_scatter0()

        for g in range(CH0 // L):
            sdix0[pl.ds(g * L, L)] = mdX[pl.ds(g * L, L)]
        _edges(0, CH0)
        pltpu.async_copy(msgbuf.at[pl.ds(0, CH0)], acc.at[sdix0],
                         semS0, add=True)

        @pl.when(wait_prev)
        def _():
            _wait_scatter1()

        for g in range(CH1 // L):
            sdix1[pl.ds(g * L, L)] = mdX[pl.ds(CH0 + g * L, L)]
        _edges(CH0, C)
        pltpu.async_copy(msgbuf.at[pl.ds(CH0, CH1)], acc.at[sdix1],
                         semS1, add=True)

    bufsA = (ijbufA, adjA, rxA, ryA, mdA, hbufA, semA, semIA)
    bufsB = (ijbufB, adjB, rxB, ryB, mdB, hbufB, semB, semIB)

    _issue_idx(0, ijbufA, semIA)
    _issue_idx(1, ijbufB, semIB)
    _prefetch(0, *bufsA)

    def _pair(k, _):
        j = 2 * k
        _prefetch(j + 1, *bufsB)
        _compute(k > 0, *bufsA)
        _prefetch(j + 2, *bufsA)
        _compute(jnp.bool_(True), *bufsB)
        return _

    lax.fori_loop(0, (NCHUNK - 1) // 2, _pair, 0, unroll=False)
    _compute(jnp.bool_(True), *bufsA)
    _wait_scatter0()
    _wait_scatter1()
    _wait_idx(ijbufA, semIA)
    _wait_idx(ijbufB, semIB)

    plsc.subcore_barrier()
    pltpu.sync_copy(acc.at[pl.ds(base_n, STRIP)],
                    aflat.at[pl.ds(cN + base_n, STRIP)])

    @pl.when(s == NS - 1)
    def _tail():
        tail = NS * STRIP
        pltpu.sync_copy(acc.at[pl.ds(tail, N - tail)],
                        aflat.at[pl.ds(cN + tail, N - tail)])


CH0 = 48
CH1 = C - CH0

QBITS = 14
QSCALE = 1 << QBITS


def _sc_aggregate(hflat, eidx, ptab, w_pos, b_pos):
    mesh = plsc.VectorSubcoreMesh(core_axis_name="c", subcore_axis_name="s",
                                  num_cores=NC, num_subcores=NS)
    f32, i32 = jnp.float32, jnp.int32
    kern = pl.kernel(
        _sc_body,
        out_type=jax.ShapeDtypeStruct((NC * N, HH), f32),
        mesh=mesh,
        scratch_types=[
            pltpu.VMEM((2, H), f32),
            pltpu.VMEM((H,), f32),
            pltpu.VMEM((N,), i32),
            pltpu.VMEM((2 * C,), i32),
            pltpu.VMEM((C,), i32),
            pltpu.VMEM((C,), f32),
            pltpu.VMEM((C,), f32),
            pltpu.VMEM((C,), i32),
            pltpu.VMEM((2 * C,), i32),
            pltpu.VMEM((C,), i32),
            pltpu.VMEM((C,), f32),
            pltpu.VMEM((C,), f32),
            pltpu.VMEM((C,), i32),
            pltpu.VMEM((CH0,), i32),
            pltpu.VMEM((CH1,), i32),
            pltpu.VMEM((C, HH), f32),
            pltpu.VMEM((C, HH), f32),
            pltpu.VMEM((C, HH), f32),
            pltpu.VMEM_SHARED((N + NS, HH), f32),
            pltpu.SemaphoreType.DMA,
            pltpu.SemaphoreType.DMA,
            pltpu.SemaphoreType.DMA,
            pltpu.SemaphoreType.DMA,
            pltpu.SemaphoreType.DMA,
            pltpu.SemaphoreType.DMA,
        ],
        compiler_params=pltpu.CompilerParams(needs_layout_passes=False),
    )
    return kern(hflat, eidx, ptab, w_pos, b_pos)


def kernel(x, edge_index, pos, region, W_pos, b_pos, W_lin, W_out, b_out):
    hflat, ptab = _matmul_a(x, W_lin, pos, region)
    aflat = _sc_aggregate(hflat, edge_index.reshape(2 * E),
                          ptab.reshape(N), W_pos, b_pos)
    return _matmul_c(aflat, W_out, b_out.reshape(1, D_OUT))


# speedup vs baseline: 1.0868x; 1.0868x over previous
import functools

import jax
import jax.numpy as jnp
from jax import lax
from jax.experimental import pallas as pl
from jax.experimental.pallas import tpu as pltpu
from jax.experimental.pallas import tpu_sc as plsc

N = 10000
E = 160000
COORS = 2
D_IN = 256
H = 256
D_OUT = 256
NC = 2
NS = 16
L = 16
HH = H // NC
EPW = E // NS
C = 80
NCHUNK = EPW // C
STRIP = 624
VPE = HH // L


_BM = 2000


def _mm_a_body(x_ref, w_ref, o_ref):
    o_ref[...] = jnp.dot(x_ref[...], w_ref[...],
                         preferred_element_type=jnp.float32)


def _matmul_a(x, w_lin):
    nb = N // _BM
    return pl.pallas_call(
        _mm_a_body,
        grid=(nb, NC),
        in_specs=[
            pl.BlockSpec((_BM, D_IN), lambda i, h: (i, 0)),
            pl.BlockSpec((D_IN, HH), lambda i, h: (0, h)),
        ],
        out_specs=pl.BlockSpec((_BM, HH), lambda i, h: (h * nb + i, 0)),
        out_shape=jax.ShapeDtypeStruct((NC * N, HH), jnp.float32),
    )(x, w_lin)


def _mm_c_body(a0_ref, a1_ref, w0_ref, w1_ref, b_ref, o_ref):
    acc = jnp.dot(a0_ref[...], w0_ref[...], preferred_element_type=jnp.float32)
    acc += jnp.dot(a1_ref[...], w1_ref[...], preferred_element_type=jnp.float32)
    o_ref[...] = acc + b_ref[...]


def _matmul_c(aflat, w_out, b_out2):
    nb = N // _BM
    return pl.pallas_call(
        _mm_c_body,
        grid=(nb,),
        in_specs=[
            pl.BlockSpec((_BM, HH), lambda i: (i, 0)),
            pl.BlockSpec((_BM, HH), lambda i: (nb + i, 0)),
            pl.BlockSpec((HH, D_OUT), lambda i: (0, 0)),
            pl.BlockSpec((HH, D_OUT), lambda i: (1, 0)),
            pl.BlockSpec((1, D_OUT), lambda i: (0, 0)),
        ],
        out_specs=pl.BlockSpec((_BM, D_OUT), lambda i: (i, 0)),
        out_shape=jax.ShapeDtypeStruct((N, D_OUT), jnp.float32),
    )(aflat, aflat, w_out, w_out, b_out2)


def _splat(v):
    return lax.broadcast(v, (L,))


def _sc_body(hflat, eidx_hbm, ptab_hbm,
             wpos_hbm, bpos_hbm,
             aflat,
             wpos_v, bpos_v, ptab_v,
             ijbufA, adjA, rxA, ryA, mdA,
             ijbufB, adjB, rxB, ryB, mdB,
             sdix0, sdix1, hbufA, hbufB, msgbuf, acc,
             semA, semB, semS0, semS1, semIA, semIB):
    c = lax.axis_index("c")
    s = lax.axis_index("s")
    cN = c * N

    pltpu.sync_copy(wpos_hbm, wpos_v)
    pltpu.sync_copy(bpos_hbm, bpos_v)
    pltpu.sync_copy(ptab_hbm, ptab_v)

    coff = c * HH
    w0 = [wpos_v[0, pl.ds(coff + v * L, L)] for v in range(VPE)]
    w1 = [wpos_v[1, pl.ds(coff + v * L, L)] for v in range(VPE)]
    bb = [bpos_v[pl.ds(coff + v * L, L)] for v in range(VPE)]

    zero16 = jnp.zeros((L,), jnp.float32)

    def _zero_msg(r, _):
        for v in range(VPE):
            msgbuf[r, pl.ds(v * L, L)] = zero16
        return _

    lax.fori_loop(0, C, _zero_msg, 0, unroll=False)
    base_n = s * STRIP
    for r in range(8):
        pltpu.sync_copy(msgbuf, acc.at[pl.ds(base_n + r * C, C)])
    plsc.subcore_barrier()

    ebase = s * EPW
    inv_q = jnp.float32(1.0 / QSCALE)

    def _wait_scatter0():
        pltpu.make_async_copy(msgbuf.at[pl.ds(0, CH0)],
                              acc.at[sdix0], semS0).wait()

    def _wait_scatter1():
        pltpu.make_async_copy(msgbuf.at[pl.ds(CH0, CH1)],
                              acc.at[sdix1], semS1).wait()

    def _issue_idx(j, ijbufX, semIX):
        off = s * EPW + jnp.minimum(j, NCHUNK - 1) * C
        pltpu.async_copy(eidx_hbm.at[pl.ds(off, C)],
                         ijbufX.at[pl.ds(0, C)], semIX)
        pltpu.async_copy(eidx_hbm.at[pl.ds(E + off, C)],
                         ijbufX.at[pl.ds(C, C)], semIX)

    def _wait_idx(ijbufX, semIX):
        pltpu.make_async_copy(eidx_hbm.at[pl.ds(0, 2 * C)], ijbufX,
                              semIX).wait()

    def _prefetch(j, ijbufX, adjX, rxX, ryX, mdX, hbufX, semX, semIX):
        _wait_idx(ijbufX, semIX)
        for g in range(C // L):
            sl = pl.ds(g * L, L)
            adjX[sl] = ijbufX[sl] + cN
        pltpu.async_copy(hflat.at[adjX], hbufX, semX)
        for g in range(C // L):
            sl = pl.ds(g * L, L)
            ps = plsc.load_gather(ptab_v, [ijbufX[sl]])
            pd = plsc.load_gather(ptab_v, [ijbufX[pl.ds(C + g * L, L)]])
            qxs = lax.shift_right_logical(ps, 18)
            qxd = lax.shift_right_logical(pd, 18)
            qys = lax.shift_right_logical(ps, 4) & 0x3FFF
            qyd = lax.shift_right_logical(pd, 4) & 0x3FFF
            rxX[sl] = (qxs - qxd).astype(jnp.float32) * inv_q
            ryX[sl] = (qys - qyd).astype(jnp.float32) * inv_q
            mdX[sl] = jnp.where((ps & 0xF) == (pd & 0xF),
                                ijbufX[pl.ds(C + g * L, L)], N + s)
        _issue_idx(j + 2, ijbufX, semIX)

    def _compute(wait_prev, ijbufX, adjX, rxX, ryX, mdX, hbufX, semX, semIX):
        pltpu.make_async_copy(hflat.at[adjX], hbufX, semX).wait()

        def _edges(lo, hi):
            @plsc.parallel_loop(lo, hi, unroll=4)
            def _edge(e):
                ev = _splat(e)
                rxv = plsc.load_gather(rxX, [ev])
                ryv = plsc.load_gather(ryX, [ev])
                for v in range(VPE):
                    sl = pl.ds(v * L, L)
                    h16 = hbufX[e, sl]
                    z = rxv * w0[v] + ryv * w1[v] + bb[v]
                    gate = jnp.maximum(z, 0.0)
                    msgbuf[e, sl] = gate * h16

        @pl.when(wait_prev)
        def _():
            _wait_scatter0()

        for g in range(CH0 // L):
            sdix0[pl.ds(g * L, L)] = mdX[pl.ds(g * L, L)]
        _edges(0, CH0)
        pltpu.async_copy(msgbuf.at[pl.ds(0, CH0)], acc.at[sdix0],
                         semS0, add=True)

        @pl.when(wait_prev)
        def _():
            _wait_scatter1()

        for g in range(CH1 // L):
            sdix1[pl.ds(g * L, L)] = mdX[pl.ds(CH0 + g * L, L)]
        _edges(CH0, C)
        pltpu.async_copy(msgbuf.at[pl.ds(CH0, CH1)], acc.at[sdix1],
                         semS1, add=True)

    bufsA = (ijbufA, adjA, rxA, ryA, mdA, hbufA, semA, semIA)
    bufsB = (ijbufB, adjB, rxB, ryB, mdB, hbufB, semB, semIB)

    _issue_idx(0, ijbufA, semIA)
    _issue_idx(1, ijbufB, semIB)
    _prefetch(0, *bufsA)

    def _pair(k, _):
        j = 2 * k
        _prefetch(j + 1, *bufsB)
        _compute(k > 0, *bufsA)
        _prefetch(j + 2, *bufsA)
        _compute(jnp.bool_(True), *bufsB)
        return _

    lax.fori_loop(0, (NCHUNK - 1) // 2, _pair, 0, unroll=False)
    _compute(jnp.bool_(True), *bufsA)
    _wait_scatter0()
    _wait_scatter1()
    _wait_idx(ijbufA, semIA)
    _wait_idx(ijbufB, semIB)

    plsc.subcore_barrier()
    pltpu.sync_copy(acc.at[pl.ds(base_n, STRIP)],
                    aflat.at[pl.ds(cN + base_n, STRIP)])

    @pl.when(s == NS - 1)
    def _tail():
        tail = NS * STRIP
        pltpu.sync_copy(acc.at[pl.ds(tail, N - tail)],
                        aflat.at[pl.ds(cN + tail, N - tail)])


CH0 = 48
CH1 = C - CH0

QBITS = 14
QSCALE = 1 << QBITS


def _sc_aggregate(hflat, eidx, ptab, w_pos, b_pos):
    mesh = plsc.VectorSubcoreMesh(core_axis_name="c", subcore_axis_name="s",
                                  num_cores=NC, num_subcores=NS)
    f32, i32 = jnp.float32, jnp.int32
    kern = pl.kernel(
        _sc_body,
        out_type=jax.ShapeDtypeStruct((NC * N, HH), f32),
        mesh=mesh,
        scratch_types=[
            pltpu.VMEM((2, H), f32),
            pltpu.VMEM((H,), f32),
            pltpu.VMEM((N,), i32),
            pltpu.VMEM((2 * C,), i32),
            pltpu.VMEM((C,), i32),
            pltpu.VMEM((C,), f32),
            pltpu.VMEM((C,), f32),
            pltpu.VMEM((C,), i32),
            pltpu.VMEM((2 * C,), i32),
            pltpu.VMEM((C,), i32),
            pltpu.VMEM((C,), f32),
            pltpu.VMEM((C,), f32),
            pltpu.VMEM((C,), i32),
            pltpu.VMEM((CH0,), i32),
            pltpu.VMEM((CH1,), i32),
            pltpu.VMEM((C, HH), f32),
            pltpu.VMEM((C, HH), f32),
            pltpu.VMEM((C, HH), f32),
            pltpu.VMEM_SHARED((N + NS, HH), f32),
            pltpu.SemaphoreType.DMA,
            pltpu.SemaphoreType.DMA,
            pltpu.SemaphoreType.DMA,
            pltpu.SemaphoreType.DMA,
            pltpu.SemaphoreType.DMA,
            pltpu.SemaphoreType.DMA,
        ],
        compiler_params=pltpu.CompilerParams(needs_layout_passes=False),
    )
    return kern(hflat, eidx, ptab, w_pos, b_pos)


def kernel(x, edge_index, pos, region, W_pos, b_pos, W_lin, W_out, b_out):
    hflat = _matmul_a(x, W_lin)
    qx = jnp.clip((pos[:, 0] * QSCALE).astype(jnp.int32), 0, QSCALE - 1)
    qy = jnp.clip((pos[:, 1] * QSCALE).astype(jnp.int32), 0, QSCALE - 1)
    ptab = (qx << 18) | (qy << 4) | (region & 0xF)
    aflat = _sc_aggregate(hflat, edge_index.reshape(2 * E), ptab,
                          W_pos, b_pos)
    return _matmul_c(aflat, W_out, b_out.reshape(1, D_OUT))
